# baseline (speedup 1.0000x reference)
import functools

import jax
import jax.numpy as jnp
from jax import lax
from jax.experimental import pallas as pl
from jax.experimental.pallas import tpu as pltpu
from jax.experimental.pallas import tpu_sc as plsc


def _sc_gather(table, idx, num_rows):
    info = plsc.get_sparse_core_info()
    nc, ns = info.num_cores, info.num_subcores
    nw = nc * ns
    d = table.shape[1]
    rows_per_w = num_rows // nw
    mesh = plsc.VectorSubcoreMesh(core_axis_name="c", subcore_axis_name="s")

    @functools.partial(
        pl.kernel,
        mesh=mesh,
        out_type=jax.ShapeDtypeStruct((num_rows, d), jnp.float32),
        scratch_types=[
            pltpu.VMEM((rows_per_w,), jnp.int32),
            pltpu.VMEM((rows_per_w, d), jnp.float32),
            pltpu.SemaphoreType.DMA,
        ],
    )
    def gather_k(table_hbm, idx_hbm, out_hbm, idx_v, rows_v, sem):
        wid = lax.axis_index("s") * nc + lax.axis_index("c")
        base = wid * rows_per_w
        pltpu.sync_copy(idx_hbm.at[pl.ds(base, rows_per_w)], idx_v)
        pltpu.async_copy(table_hbm.at[idx_v], rows_v, sem).wait()
        pltpu.sync_copy(rows_v, out_hbm.at[pl.ds(base, rows_per_w)])

    return gather_k(table, idx)


def kernel(batch_nodes, cflat, bflat, embeddings, weights, cls_w, cls_b):
    n, e = embeddings.shape
    r = weights.shape[1]
    b = batch_nodes.shape[0]
    ncls = cls_w.shape[0]
    nnz = cflat.shape[0]
    p = ((nnz + 255) // 256) * 256
    pad = p - nnz

    s_flat = jnp.pad(cflat[:, 0], (0, pad), constant_values=-1)
    rel_flat = jnp.pad(cflat[:, 1], (0, pad), constant_values=-1)
    o_flat = jnp.pad(cflat[:, 2], (0, pad), constant_values=0)
    b_flat = jnp.pad(bflat, (0, pad), constant_values=-1)

    rows_per_gather = 128 // e
    nq = n // rows_per_gather
    table128 = jnp.concatenate(
        [embeddings[a * nq:(a + 1) * nq] for a in range(rows_per_gather)],
        axis=1)
    e_rows = _sc_gather(table128, o_flat % nq, p)

    s_c, rel_c, o_c, bx_c = (a.reshape(p, 1)
                             for a in (s_flat, rel_flat, o_flat, b_flat))
    s_r, rel_r, bx_r = (a.reshape(1, p) for a in (s_flat, rel_flat, b_flat))
    bn_c = batch_nodes.reshape(b, 1)

    w0s = weights[0].transpose(0, 2, 1).reshape(r * e, e)
    w1s = weights[1].transpose(0, 2, 1).reshape(r * e, e)
    cls_wt = cls_w.T
    cls_b2 = cls_b.reshape(1, ncls)

    def body(sc_ref, relc_ref, oc_ref, bxc_ref, sr_ref, relr_ref, bxr_ref,
             e_ref, bn_ref, w0_ref, w1_ref, cw_ref, cb_ref, out_ref):
        s_col, rel_col, o_col, bx_col = (
            sc_ref[:], relc_ref[:], oc_ref[:], bxc_ref[:])
        s_row, rel_row, bx_row = sr_ref[:], relr_ref[:], bxr_ref[:]
        g128 = e_ref[:]
        osub = lax.div(o_col, nq)
        erows = g128[:, 0:e] * (osub == 0).astype(jnp.float32)
        for k in range(1, rows_per_gather):
            erows = erows + (g128[:, k * e:(k + 1) * e]
                             * (osub == k).astype(jnp.float32))
        fr_col = s_col + n * rel_col + (n * r) * bx_col
        fr_row = s_row + n * rel_row + (n * r) * bx_row
        eqf = (fr_col == fr_row).astype(jnp.float32)
        vals = 1.0 / jnp.sum(eqf, axis=1, keepdims=True)
        ev = erows * vals
        stk1 = jnp.concatenate(
            [jnp.where(rel_col == rr, ev, 0.0) for rr in range(r)], axis=1)
        t1 = jnp.dot(stk1, w0_ref[:], preferred_element_type=jnp.float32)
        skey_row = bx_row * n + s_row
        qkey_col = bx_col * n + o_col
        m2 = (qkey_col == skey_row).astype(jnp.float32)
        g2 = jnp.maximum(
            jnp.dot(m2, t1, preferred_element_type=jnp.float32), 0.0)
        stk2 = jnp.concatenate(
            [jnp.where(rel_col == rr, g2 * vals, 0.0) for rr in range(r)],
            axis=1)
        t2 = jnp.dot(stk2, w1_ref[:], preferred_element_type=jnp.float32)
        bn_col = bn_ref[:]
        bmask = lax.broadcasted_iota(jnp.int32, (b, p), 0) == bx_row
        bn_of_row = jnp.sum(jnp.where(bmask, bn_col, 0), axis=0,
                            keepdims=True)
        bsel_row = jnp.where(s_row == bn_of_row, bx_row, -1)
        mout = (lax.broadcasted_iota(jnp.int32, (b, p), 0)
                == bsel_row).astype(jnp.float32)
        acc = jnp.dot(mout, t2, preferred_element_type=jnp.float32)
        pooled = jnp.maximum(acc, 0.0)
        out_ref[:] = (jnp.dot(pooled, cw_ref[:],
                              preferred_element_type=jnp.float32) + cb_ref[:])

    logits = pl.pallas_call(
        body,
        out_shape=jax.ShapeDtypeStruct((b, ncls), jnp.float32),
        compiler_params=pltpu.CompilerParams(
            vmem_limit_bytes=128 * 1024 * 1024),
    )(s_c, rel_c, o_c, bx_c, s_r, rel_r, bx_r, e_rows, bn_c,
      w0s, w1s, cls_wt, cls_b2)
    return logits


# speedup vs baseline: 1.2208x; 1.2208x over previous
import functools

import jax
import jax.numpy as jnp
from jax import lax
from jax.experimental import pallas as pl
from jax.experimental.pallas import tpu as pltpu
from jax.experimental.pallas import tpu_sc as plsc


def _sc_gather(table, idx, num_rows):
    info = plsc.get_sparse_core_info()
    nc, ns = info.num_cores, info.num_subcores
    nw = nc * ns
    d = table.shape[1]
    rows_per_w = num_rows // nw
    mesh = plsc.VectorSubcoreMesh(core_axis_name="c", subcore_axis_name="s")

    @functools.partial(
        pl.kernel,
        mesh=mesh,
        out_type=jax.ShapeDtypeStruct((num_rows, d), jnp.float32),
        scratch_types=[
            pltpu.VMEM((rows_per_w,), jnp.int32),
            pltpu.VMEM((rows_per_w, d), jnp.float32),
            pltpu.SemaphoreType.DMA,
        ],
    )
    def gather_k(table_hbm, idx_hbm, out_hbm, idx_v, rows_v, sem):
        wid = lax.axis_index("s") * nc + lax.axis_index("c")
        base = wid * rows_per_w
        pltpu.sync_copy(idx_hbm.at[pl.ds(base, rows_per_w)], idx_v)
        pltpu.async_copy(table_hbm.at[idx_v], rows_v, sem).wait()
        pltpu.sync_copy(rows_v, out_hbm.at[pl.ds(base, rows_per_w)])

    return gather_k(table, idx)


def kernel(batch_nodes, cflat, bflat, embeddings, weights, cls_w, cls_b):
    n, e = embeddings.shape
    r = weights.shape[1]
    b = batch_nodes.shape[0]
    ncls = cls_w.shape[0]
    nnz = cflat.shape[0]
    p = ((nnz + 255) // 256) * 256
    pad = p - nnz

    s_flat = jnp.pad(cflat[:, 0], (0, pad), constant_values=-1)
    rel_flat = jnp.pad(cflat[:, 1], (0, pad), constant_values=-1)
    o_flat = jnp.pad(cflat[:, 2], (0, pad), constant_values=0)
    b_flat = jnp.pad(bflat, (0, pad), constant_values=-1)

    rows_per_gather = 128 // e
    nq = n // rows_per_gather

    def trans_body(x_ref, out_ref):
        x = x_ref[:]
        for a in range(rows_per_gather):
            out_ref[:, a * e:(a + 1) * e] = x[:, a * nq:(a + 1) * nq].T

    table128 = pl.pallas_call(
        trans_body,
        out_shape=jax.ShapeDtypeStruct((nq, 128), jnp.float32),
        compiler_params=pltpu.CompilerParams(
            vmem_limit_bytes=128 * 1024 * 1024),
    )(embeddings.T)
    e_rows = _sc_gather(table128, o_flat % nq, p)

    s_c, rel_c, o_c, bx_c = (a.reshape(p, 1)
                             for a in (s_flat, rel_flat, o_flat, b_flat))
    s_r, rel_r, bx_r = (a.reshape(1, p) for a in (s_flat, rel_flat, b_flat))
    bn_c = batch_nodes.reshape(b, 1)

    w0s = weights[0].transpose(0, 2, 1).reshape(r * e, e)
    w1s = weights[1].transpose(0, 2, 1).reshape(r * e, e)
    cls_wt = cls_w.T
    cls_b2 = cls_b.reshape(1, ncls)

    def body(sc_ref, relc_ref, oc_ref, bxc_ref, sr_ref, relr_ref, bxr_ref,
             e_ref, bn_ref, w0_ref, w1_ref, cw_ref, cb_ref, out_ref):
        s_col, rel_col, o_col, bx_col = (
            sc_ref[:], relc_ref[:], oc_ref[:], bxc_ref[:])
        s_row, rel_row, bx_row = sr_ref[:], relr_ref[:], bxr_ref[:]
        g128 = e_ref[:]
        osub = lax.div(o_col, nq)
        erows = g128[:, 0:e] * (osub == 0).astype(jnp.float32)
        for k in range(1, rows_per_gather):
            erows = erows + (g128[:, k * e:(k + 1) * e]
                             * (osub == k).astype(jnp.float32))
        fr_col = s_col + n * rel_col + (n * r) * bx_col
        fr_row = s_row + n * rel_row + (n * r) * bx_row
        eqf = (fr_col == fr_row).astype(jnp.float32)
        vals = 1.0 / jnp.sum(eqf, axis=1, keepdims=True)
        ev = erows * vals
        stk1 = jnp.concatenate(
            [jnp.where(rel_col == rr, ev, 0.0) for rr in range(r)], axis=1)
        t1 = jnp.dot(stk1, w0_ref[:], preferred_element_type=jnp.float32)
        skey_row = bx_row * n + s_row
        qkey_col = bx_col * n + o_col
        m2 = (qkey_col == skey_row).astype(jnp.float32)
        g2 = jnp.maximum(
            jnp.dot(m2, t1, preferred_element_type=jnp.float32), 0.0)
        stk2 = jnp.concatenate(
            [jnp.where(rel_col == rr, g2 * vals, 0.0) for rr in range(r)],
            axis=1)
        t2 = jnp.dot(stk2, w1_ref[:], preferred_element_type=jnp.float32)
        bn_col = bn_ref[:]
        bmask = lax.broadcasted_iota(jnp.int32, (b, p), 0) == bx_row
        bn_of_row = jnp.sum(jnp.where(bmask, bn_col, 0), axis=0,
                            keepdims=True)
        bsel_row = jnp.where(s_row == bn_of_row, bx_row, -1)
        mout = (lax.broadcasted_iota(jnp.int32, (b, p), 0)
                == bsel_row).astype(jnp.float32)
        acc = jnp.dot(mout, t2, preferred_element_type=jnp.float32)
        pooled = jnp.maximum(acc, 0.0)
        out_ref[:] = (jnp.dot(pooled, cw_ref[:],
                              preferred_element_type=jnp.float32) + cb_ref[:])

    logits = pl.pallas_call(
        body,
        out_shape=jax.ShapeDtypeStruct((b, ncls), jnp.float32),
        compiler_params=pltpu.CompilerParams(
            vmem_limit_bytes=128 * 1024 * 1024),
    )(s_c, rel_c, o_c, bx_c, s_r, rel_r, bx_r, e_rows, bn_c,
      w0s, w1s, cls_wt, cls_b2)
    return logits


# speedup vs baseline: 1.2532x; 1.0266x over previous
import functools

import jax
import jax.numpy as jnp
from jax import lax
from jax.experimental import pallas as pl
from jax.experimental.pallas import tpu as pltpu
from jax.experimental.pallas import tpu_sc as plsc


def _sc_gather(table, idx, num_rows):
    info = plsc.get_sparse_core_info()
    nc, ns = info.num_cores, info.num_subcores
    nw = nc * ns
    d = table.shape[1]
    rows_per_w = num_rows // nw
    mesh = plsc.VectorSubcoreMesh(core_axis_name="c", subcore_axis_name="s")

    @functools.partial(
        pl.kernel,
        mesh=mesh,
        out_type=jax.ShapeDtypeStruct((num_rows, d), jnp.float32),
        scratch_types=[
            pltpu.VMEM((rows_per_w,), jnp.int32),
            pltpu.VMEM((rows_per_w, d), jnp.float32),
            pltpu.SemaphoreType.DMA,
        ],
    )
    def gather_k(table_hbm, idx_hbm, out_hbm, idx_v, rows_v, sem):
        wid = lax.axis_index("s") * nc + lax.axis_index("c")
        base = wid * rows_per_w
        pltpu.sync_copy(idx_hbm.at[pl.ds(base, rows_per_w)], idx_v)
        pltpu.async_copy(table_hbm.at[idx_v], rows_v, sem).wait()
        pltpu.sync_copy(rows_v, out_hbm.at[pl.ds(base, rows_per_w)])

    return gather_k(table, idx)


def kernel(batch_nodes, cflat, bflat, embeddings, weights, cls_w, cls_b):
    n, e = embeddings.shape
    r = weights.shape[1]
    b = batch_nodes.shape[0]
    ncls = cls_w.shape[0]
    nnz = cflat.shape[0]
    p = ((nnz + 255) // 256) * 256
    pad = p - nnz

    s_flat = jnp.pad(cflat[:, 0], (0, pad), constant_values=-1)
    rel_flat = jnp.pad(cflat[:, 1], (0, pad), constant_values=-1)
    o_flat = jnp.pad(cflat[:, 2], (0, pad), constant_values=0)
    b_flat = jnp.pad(bflat, (0, pad), constant_values=-1)

    rows_per_gather = 128 // e
    nq = n // rows_per_gather

    def trans_body(x_ref, out_ref):
        x = x_ref[:]
        for a in range(rows_per_gather):
            out_ref[:, a * e:(a + 1) * e] = x[:, a * nq:(a + 1) * nq].T

    table128 = pl.pallas_call(
        trans_body,
        out_shape=jax.ShapeDtypeStruct((nq, 128), jnp.float32),
        compiler_params=pltpu.CompilerParams(
            vmem_limit_bytes=128 * 1024 * 1024),
    )(embeddings.T)
    e_rows = _sc_gather(table128, o_flat % nq, p)

    s_c, rel_c, o_c, bx_c = (a.reshape(p, 1)
                             for a in (s_flat, rel_flat, o_flat, b_flat))
    s_r, rel_r, bx_r = (a.reshape(1, p) for a in (s_flat, rel_flat, b_flat))
    bn_c = batch_nodes.reshape(b, 1)

    w0s = weights[0].transpose(0, 2, 1).reshape(r * e, e)
    w1s = weights[1].transpose(0, 2, 1).reshape(r * e, e)
    cls_wt = cls_w.T
    cls_b2 = cls_b.reshape(1, ncls)
    sel = (jnp.arange(128)[:, None] % e
           == jnp.arange(e)[None, :]).astype(jnp.float32)

    def body(sc_ref, relc_ref, oc_ref, bxc_ref, sr_ref, relr_ref, bxr_ref,
             e_ref, bn_ref, w0_ref, w1_ref, cw_ref, cb_ref, sel_ref,
             out_ref):
        s_col, rel_col, o_col, bx_col = (
            sc_ref[:], relc_ref[:], oc_ref[:], bxc_ref[:])
        s_row, rel_row, bx_row = sr_ref[:], relr_ref[:], bxr_ref[:]
        g128 = e_ref[:]
        osub = lax.div(o_col, nq)
        lane_blk = lax.broadcasted_iota(jnp.int32, (p, 128), 1) // e
        gm = jnp.where(lane_blk == osub, g128, 0.0)
        erows = jnp.dot(gm, sel_ref[:], preferred_element_type=jnp.float32)
        fr_col = s_col + n * rel_col + (n * r) * bx_col
        fr_row = s_row + n * rel_row + (n * r) * bx_row
        eqf = (fr_col == fr_row).astype(jnp.float32)
        vals = 1.0 / jnp.sum(eqf, axis=1, keepdims=True)
        ev = erows * vals
        stk1 = jnp.concatenate(
            [jnp.where(rel_col == rr, ev, 0.0) for rr in range(r)], axis=1)
        t1 = jnp.dot(stk1, w0_ref[:], preferred_element_type=jnp.float32)
        skey_row = bx_row * n + s_row
        qkey_col = bx_col * n + o_col
        m2 = (qkey_col == skey_row).astype(jnp.float32)
        g2 = jnp.maximum(
            jnp.dot(m2, t1, preferred_element_type=jnp.float32), 0.0)
        stk2 = jnp.concatenate(
            [jnp.where(rel_col == rr, g2 * vals, 0.0) for rr in range(r)],
            axis=1)
        t2 = jnp.dot(stk2, w1_ref[:], preferred_element_type=jnp.float32)
        bn_col = bn_ref[:]
        bmask = lax.broadcasted_iota(jnp.int32, (b, p), 0) == bx_row
        bn_of_row = jnp.sum(jnp.where(bmask, bn_col, 0), axis=0,
                            keepdims=True)
        bsel_row = jnp.where(s_row == bn_of_row, bx_row, -1)
        mout = (lax.broadcasted_iota(jnp.int32, (b, p), 0)
                == bsel_row).astype(jnp.float32)
        acc = jnp.dot(mout, t2, preferred_element_type=jnp.float32)
        pooled = jnp.maximum(acc, 0.0)
        out_ref[:] = (jnp.dot(pooled, cw_ref[:],
                              preferred_element_type=jnp.float32) + cb_ref[:])

    logits = pl.pallas_call(
        body,
        out_shape=jax.ShapeDtypeStruct((b, ncls), jnp.float32),
        compiler_params=pltpu.CompilerParams(
            vmem_limit_bytes=128 * 1024 * 1024),
    )(s_c, rel_c, o_c, bx_c, s_r, rel_r, bx_r, e_rows, bn_c,
      w0s, w1s, cls_wt, cls_b2, sel)
    return logits
